# baseline (speedup 1.0000x reference)
import functools

import jax
import jax.numpy as jnp
from jax import lax
from jax.experimental import pallas as pl
from jax.experimental.pallas import tpu as pltpu
from jax.experimental.pallas import tpu_sc as plsc

N_LAYERS_ = 10
BATCH_ = 32
CHANNELS_ = 1024
L_CACHE_ = 20
LAYER_IDX_ = 0
LANES_ = 16
CH_ = CHANNELS_ // 128


def _sc_body(cp_hbm, a_hbm, w_hbm, bx_hbm, out_hbm,
             cp_v, a_v, w_v, bx_v, out_v):
    b = lax.axis_index("s") * 2 + lax.axis_index("c")
    b_hi = b // 8
    b_lo = lax.rem(b, 8)
    pltpu.sync_copy(cp_hbm, cp_v)
    for m in range(L_CACHE_):
        pltpu.sync_copy(a_hbm.at[LAYER_IDX_, m, b_hi, :, b_lo, :],
                        a_v.at[m])
    pltpu.sync_copy(w_hbm, w_v)
    pltpu.sync_copy(bx_hbm.at[b], bx_v)
    cp = jnp.clip(cp_v[...], 0, L_CACHE_ - 1)
    cpp = lax.rem(cp + 1, L_CACHE_)

    @pl.loop(0, 128 // LANES_)
    def _chunk(j):
        sl = pl.ds(j * LANES_, LANES_)
        for ch in range(CH_):
            bx = bx_v[ch, sl]
            acc = jnp.zeros((LANES_,), jnp.float32)
            for m in range(L_CACHE_):
                mw = (m + L_CACHE_ - 1) % L_CACHE_
                wm = w_v[mw, ch, sl]
                am = jnp.where(cpp == m, bx, a_v[m, ch, sl])
                acc = acc + wm * am
            out_v[ch, sl] = acc

    pltpu.sync_copy(out_v, out_hbm.at[b])


def kernel(Bx, cache_position, seq_len, conv_cache, conv_weight):
    del seq_len
    at6 = jnp.transpose(conv_cache, (0, 3, 1, 2)).reshape(
        N_LAYERS_, L_CACHE_, 4, 8, 8, 128).transpose(0, 1, 2, 4, 3, 5)
    w8 = jnp.transpose(conv_weight, (1, 0)).reshape(L_CACHE_, 8, 128)
    bx8 = jnp.reshape(Bx, (BATCH_, 8, 128))
    cp16 = jnp.broadcast_to(cache_position, (LANES_,))
    mesh = plsc.VectorSubcoreMesh(core_axis_name="c", subcore_axis_name="s")
    sc = functools.partial(
        pl.kernel,
        out_type=jax.ShapeDtypeStruct((BATCH_, 8, 128), jnp.float32),
        mesh=mesh,
        scratch_types=[
            pltpu.VMEM((LANES_,), jnp.int32),
            pltpu.VMEM((L_CACHE_, 8, 128), jnp.float32),
            pltpu.VMEM((L_CACHE_, 8, 128), jnp.float32),
            pltpu.VMEM((8, 128), jnp.float32),
            pltpu.VMEM((8, 128), jnp.float32),
        ],
    )(_sc_body)
    out = sc(cp16, at6, w8, bx8)
    return out.reshape(BATCH_, CHANNELS_, 1)
